# baseline (speedup 1.0000x reference)
import functools

import jax
import jax.numpy as jnp
from jax.experimental import pallas as pl
from jax.experimental.pallas import tpu as pltpu

MAX_LEN = 4096
D_MODEL = 1024
BATCH = 4

CHUNK_ROWS = 256
NUM_CHUNKS = MAX_LEN // CHUNK_ROWS


def _dma_body(table_hbm, out_hbm, buf, rsems, wsems):
    def read(c):
        h = pltpu.make_async_copy(
            table_hbm.at[pl.ds(c * CHUNK_ROWS, CHUNK_ROWS), :],
            buf.at[c], rsems.at[c])
        h.start()
        return h

    def write(c, b):
        h = pltpu.make_async_copy(
            buf.at[c],
            out_hbm.at[b, pl.ds(c * CHUNK_ROWS, CHUNK_ROWS), :],
            wsems.at[c])
        h.start()
        return h

    reads = [read(c) for c in range(NUM_CHUNKS)]
    writes = []
    for c in range(NUM_CHUNKS):
        reads[c].wait()
        writes += [write(c, b) for b in range(BATCH)]
    for h in writes:
        h.wait()


@jax.jit
def _broadcast_table(emb_weight):
    return pl.pallas_call(
        _dma_body,
        in_specs=[pl.BlockSpec(memory_space=pltpu.MemorySpace.HBM)],
        out_specs=pl.BlockSpec(memory_space=pltpu.MemorySpace.HBM),
        out_shape=jax.ShapeDtypeStruct((BATCH, MAX_LEN, D_MODEL), jnp.float32),
        scratch_shapes=[
            pltpu.VMEM((NUM_CHUNKS, CHUNK_ROWS, D_MODEL), jnp.float32),
            pltpu.SemaphoreType.DMA((NUM_CHUNKS,)),
            pltpu.SemaphoreType.DMA((NUM_CHUNKS,)),
        ],
    )(emb_weight)


def kernel(x, emb_weight):
    del x
    return _broadcast_table(emb_weight)


# speedup vs baseline: 1.0515x; 1.0515x over previous
import functools

import jax
import jax.numpy as jnp
from jax.experimental import pallas as pl
from jax.experimental.pallas import tpu as pltpu

MAX_LEN = 4096
D_MODEL = 1024
BATCH = 4

CHUNK_ROWS = 1024
NUM_CHUNKS = MAX_LEN // CHUNK_ROWS


def _dma_body(table_hbm, out_hbm, buf, rsems, wsems):
    def read(c):
        h = pltpu.make_async_copy(
            table_hbm.at[pl.ds(c * CHUNK_ROWS, CHUNK_ROWS), :],
            buf.at[c], rsems.at[c])
        h.start()
        return h

    def write(c, b):
        h = pltpu.make_async_copy(
            buf.at[c],
            out_hbm.at[b, pl.ds(c * CHUNK_ROWS, CHUNK_ROWS), :],
            wsems.at[c])
        h.start()
        return h

    reads = [read(c) for c in range(NUM_CHUNKS)]
    writes = []
    for c in range(NUM_CHUNKS):
        reads[c].wait()
        writes += [write(c, b) for b in range(BATCH)]
    for h in writes:
        h.wait()


@jax.jit
def _broadcast_table(emb_weight):
    return pl.pallas_call(
        _dma_body,
        in_specs=[pl.BlockSpec(memory_space=pltpu.MemorySpace.HBM)],
        out_specs=pl.BlockSpec(memory_space=pltpu.MemorySpace.HBM),
        out_shape=jax.ShapeDtypeStruct((BATCH, MAX_LEN, D_MODEL), jnp.float32),
        scratch_shapes=[
            pltpu.VMEM((NUM_CHUNKS, CHUNK_ROWS, D_MODEL), jnp.float32),
            pltpu.SemaphoreType.DMA((NUM_CHUNKS,)),
            pltpu.SemaphoreType.DMA((NUM_CHUNKS,)),
        ],
    )(emb_weight)


def kernel(x, emb_weight):
    del x
    return _broadcast_table(emb_weight)
